# Initial kernel scaffold; baseline (speedup 1.0000x reference)
import jax
import jax.numpy as jnp
from jax.experimental import pallas as pl


def kernel(x, edge_index, batch, W1, b1, W2, b2, Wfc, bfc):
    raise NotImplementedError("write your pallas kernel here")


# speedup vs baseline: 8.9728x; 8.9728x over previous
import functools

import jax
import jax.numpy as jnp
from jax import lax
from jax.experimental import pallas as pl
from jax.experimental.pallas import tpu as pltpu
from jax.experimental.pallas import tpu_sc as plsc

N_NODES = 10000
N_GRAPHS = 64
D_IN = 128
H1 = 128
H2 = 64
H2P = 128
D_OUT = 10

NC = 2
NS = 16
NW = NC * NS
CHUNK = 128
N_PAD = 10240
DUMMY = 10200
ROWS_PT = N_PAD // NS
NCHUNK = 80
E_PAD = NW * NCHUNK * CHUNK
RB = 640
NB = N_PAD // RB


def _sc_degree(dst_r):
    mesh = plsc.VectorSubcoreMesh(core_axis_name="c", subcore_axis_name="s")

    @functools.partial(
        pl.kernel,
        out_type=jax.ShapeDtypeStruct((NC, N_PAD), jnp.float32),
        mesh=mesh,
        scratch_types=[
            pltpu.VMEM((NCHUNK, CHUNK), jnp.int32),
            pltpu.VMEM((CHUNK,), jnp.float32),
            pltpu.VMEM((ROWS_PT,), jnp.float32),
            pltpu.VMEM_SHARED((N_PAD,), jnp.float32),
        ],
    )
    def deg_kernel(dst_hbm, out_hbm, idx_v, ones_v, stage_v, acc_sh):
        c = lax.axis_index("c")
        s = lax.axis_index("s")
        wid = s * NC + c

        def zfill(i, _):
            stage_v[pl.ds(i * 16, 16)] = jnp.zeros((16,), jnp.float32)
            return 0

        lax.fori_loop(0, ROWS_PT // 16, zfill, 0)

        def ofill(i, _):
            ones_v[pl.ds(i * 16, 16)] = jnp.ones((16,), jnp.float32)
            return 0

        lax.fori_loop(0, CHUNK // 16, ofill, 0)

        pltpu.sync_copy(stage_v, acc_sh.at[pl.ds(s * ROWS_PT, ROWS_PT)])
        pltpu.sync_copy(dst_hbm.at[wid], idx_v)
        plsc.subcore_barrier()

        def edge_body(j, _):
            pltpu.sync_copy(ones_v, acc_sh.at[idx_v.at[j]], add=True)
            return 0

        lax.fori_loop(0, NCHUNK, edge_body, 0)
        plsc.subcore_barrier()
        pltpu.sync_copy(acc_sh.at[pl.ds(s * ROWS_PT, ROWS_PT)], stage_v)
        pltpu.sync_copy(stage_v, out_hbm.at[c, pl.ds(s * ROWS_PT, ROWS_PT)])

    return deg_kernel(dst_r)


def _sc_spmm(p, src_r, dst_r, d):
    mesh = plsc.VectorSubcoreMesh(core_axis_name="c", subcore_axis_name="s")

    @functools.partial(
        pl.kernel,
        out_type=jax.ShapeDtypeStruct((NC, N_PAD, d), jnp.float32),
        mesh=mesh,
        scratch_types=[
            pltpu.VMEM((NCHUNK, CHUNK), jnp.int32),
            pltpu.VMEM((NCHUNK, CHUNK), jnp.int32),
            pltpu.VMEM((CHUNK, d), jnp.float32),
            pltpu.VMEM_SHARED((N_PAD, d), jnp.float32),
        ],
    )
    def spmm_kernel(p_hbm, src_hbm, dst_hbm, out_hbm, sidx, didx, rows, acc_sh):
        c = lax.axis_index("c")
        s = lax.axis_index("s")
        wid = s * NC + c

        def zrow(i, _):
            for q in range(d // 16):
                rows[i, pl.ds(q * 16, 16)] = jnp.zeros((16,), jnp.float32)
            return 0

        lax.fori_loop(0, CHUNK, zrow, 0)
        for t in range(ROWS_PT // CHUNK):
            pltpu.sync_copy(
                rows, acc_sh.at[pl.ds(s * ROWS_PT + t * CHUNK, CHUNK)])
        pltpu.sync_copy(src_hbm.at[wid], sidx)
        pltpu.sync_copy(dst_hbm.at[wid], didx)
        plsc.subcore_barrier()

        def edge_body(j, _):
            pltpu.sync_copy(p_hbm.at[sidx.at[j]], rows)
            pltpu.sync_copy(rows, acc_sh.at[didx.at[j]], add=True)
            return 0

        lax.fori_loop(0, NCHUNK, edge_body, 0)
        plsc.subcore_barrier()
        for t in range(ROWS_PT // CHUNK):
            pltpu.sync_copy(
                acc_sh.at[pl.ds(s * ROWS_PT + t * CHUNK, CHUNK)], rows)
            pltpu.sync_copy(
                rows, out_hbm.at[c, pl.ds(s * ROWS_PT + t * CHUNK, CHUNK)])

    return spmm_kernel(p, src_r, dst_r)


def _tc1(x_pad, W1, d0, d1):
    def body(x_ref, w_ref, d0_ref, d1_ref, p_ref, dis_ref):
        deg = d0_ref[...] + d1_ref[...] + 1.0
        dis = lax.rsqrt(deg)
        h = jnp.dot(x_ref[...], w_ref[...], preferred_element_type=jnp.float32)
        p_ref[...] = h * dis
        dis_ref[...] = dis

    return pl.pallas_call(
        body,
        grid=(NB,),
        in_specs=[
            pl.BlockSpec((RB, D_IN), lambda i: (i, 0)),
            pl.BlockSpec((D_IN, H1), lambda i: (0, 0)),
            pl.BlockSpec((RB, 1), lambda i: (i, 0)),
            pl.BlockSpec((RB, 1), lambda i: (i, 0)),
        ],
        out_specs=[
            pl.BlockSpec((RB, H1), lambda i: (i, 0)),
            pl.BlockSpec((RB, 1), lambda i: (i, 0)),
        ],
        out_shape=[
            jax.ShapeDtypeStruct((N_PAD, H1), jnp.float32),
            jax.ShapeDtypeStruct((N_PAD, 1), jnp.float32),
        ],
    )(x_pad, W1, d0, d1)


def _tc2(s1a, s1b, p1, dis, b1, W2):
    def body(a_ref, b_ref, p_ref, dis_ref, bias_ref, w_ref, o_ref):
        dis = dis_ref[...]
        t = jnp.maximum(
            dis * (a_ref[...] + b_ref[...] + p_ref[...]) + bias_ref[...], 0.0)
        o_ref[...] = dis * jnp.dot(
            t, w_ref[...], preferred_element_type=jnp.float32)

    return pl.pallas_call(
        body,
        grid=(NB,),
        in_specs=[
            pl.BlockSpec((RB, H1), lambda i: (i, 0)),
            pl.BlockSpec((RB, H1), lambda i: (i, 0)),
            pl.BlockSpec((RB, H1), lambda i: (i, 0)),
            pl.BlockSpec((RB, 1), lambda i: (i, 0)),
            pl.BlockSpec((1, H1), lambda i: (0, 0)),
            pl.BlockSpec((H1, H2P), lambda i: (0, 0)),
        ],
        out_specs=pl.BlockSpec((RB, H2P), lambda i: (i, 0)),
        out_shape=jax.ShapeDtypeStruct((N_PAD, H2P), jnp.float32),
    )(s1a, s1b, p1, dis, b1, W2)


def _tc3(s2a, s2b, p2, dis, b2, batch_col, Wfc, bfc):
    def body(a_ref, b_ref, p_ref, dis_ref, bias_ref, bt_ref, wfc_ref, bfc_ref,
             o_ref, sums, cnts):
        i = pl.program_id(0)

        @pl.when(i == 0)
        def _():
            sums[...] = jnp.zeros_like(sums)
            cnts[...] = jnp.zeros_like(cnts)

        dis = dis_ref[...]
        t = jnp.maximum(
            dis * (a_ref[...] + b_ref[...] + p_ref[...]) + bias_ref[...], 0.0)
        gids = lax.broadcasted_iota(jnp.int32, (RB, N_GRAPHS), 1)
        onehot = (bt_ref[...] == gids).astype(jnp.float32)
        sums[...] += lax.dot_general(
            onehot, t, (((0,), (0,)), ((), ())),
            preferred_element_type=jnp.float32,
            precision=lax.Precision.HIGHEST)
        cnts[...] += lax.dot_general(
            onehot, jnp.ones((RB, 1), jnp.float32), (((0,), (0,)), ((), ())),
            preferred_element_type=jnp.float32,
            precision=lax.Precision.HIGHEST)

        @pl.when(i == NB - 1)
        def _():
            g = sums[...] / jnp.maximum(cnts[...], 1.0)
            logits = jnp.dot(
                g, wfc_ref[...], preferred_element_type=jnp.float32,
                precision=lax.Precision.HIGHEST) + bfc_ref[...]
            m = jnp.max(logits, axis=1, keepdims=True)
            e = jnp.exp(logits - m)
            o_ref[...] = e / jnp.sum(e, axis=1, keepdims=True)

    return pl.pallas_call(
        body,
        grid=(NB,),
        in_specs=[
            pl.BlockSpec((RB, H2P), lambda i: (i, 0)),
            pl.BlockSpec((RB, H2P), lambda i: (i, 0)),
            pl.BlockSpec((RB, H2P), lambda i: (i, 0)),
            pl.BlockSpec((RB, 1), lambda i: (i, 0)),
            pl.BlockSpec((1, H2P), lambda i: (0, 0)),
            pl.BlockSpec((RB, 1), lambda i: (i, 0)),
            pl.BlockSpec((H2P, D_OUT), lambda i: (0, 0)),
            pl.BlockSpec((1, D_OUT), lambda i: (0, 0)),
        ],
        out_specs=pl.BlockSpec((N_GRAPHS, D_OUT), lambda i: (0, 0)),
        out_shape=jax.ShapeDtypeStruct((N_GRAPHS, D_OUT), jnp.float32),
        scratch_shapes=[
            pltpu.VMEM((N_GRAPHS, H2P), jnp.float32),
            pltpu.VMEM((N_GRAPHS, 1), jnp.float32),
        ],
    )(s2a, s2b, p2, dis, b2, batch_col, Wfc, bfc)


def kernel(x, edge_index, batch, W1, b1, W2, b2, Wfc, bfc):
    src = edge_index[0].astype(jnp.int32)
    dst = edge_index[1].astype(jnp.int32)
    n_edges = src.shape[0]
    fill = jnp.full((E_PAD - n_edges,), DUMMY, dtype=jnp.int32)
    src_r = jnp.concatenate([src, fill]).reshape(NW, NCHUNK, CHUNK)
    dst_r = jnp.concatenate([dst, fill]).reshape(NW, NCHUNK, CHUNK)
    x_pad = jnp.pad(x, ((0, N_PAD - N_NODES), (0, 0)))
    batch_col = jnp.pad(
        batch.astype(jnp.int32), (0, N_PAD - N_NODES),
        constant_values=N_GRAPHS).reshape(N_PAD, 1)

    deg_parts = _sc_degree(dst_r)
    d0 = deg_parts[0].reshape(N_PAD, 1)
    d1 = deg_parts[1].reshape(N_PAD, 1)

    W2p = jnp.pad(W2, ((0, 0), (0, H2P - H2)))
    b2p = jnp.pad(b2, (0, H2P - H2)).reshape(1, H2P)
    Wfcp = jnp.pad(Wfc, ((0, H2P - H2), (0, 0)))

    p1, dis = _tc1(x_pad, W1, d0, d1)
    s1 = _sc_spmm(p1, src_r, dst_r, H1)
    p2 = _tc2(s1[0], s1[1], p1, dis, b1.reshape(1, H1), W2p)
    s2 = _sc_spmm(p2, src_r, dst_r, H2P)
    return _tc3(s2[0], s2[1], p2, dis, b2p, batch_col,
                Wfcp, bfc.reshape(1, D_OUT))


# speedup vs baseline: 17.5380x; 1.9546x over previous
import functools

import jax
import jax.numpy as jnp
from jax import lax
from jax.experimental import pallas as pl
from jax.experimental.pallas import tpu as pltpu
from jax.experimental.pallas import tpu_sc as plsc

N_NODES = 10000
N_GRAPHS = 64
D_IN = 128
H1 = 128
H2 = 64
H2P = 128
D_OUT = 10

NC = 2
NS = 16
NW = NC * NS
CHUNK = 128
N_PAD = 10240
DUMMY = 10200
ROWS_PT = N_PAD // NS
NCHUNK = 80
E_PAD = NW * NCHUNK * CHUNK
CHUNK1, NCHUNK1, NBUF1 = 112, 90, 2
CHUNK2, NCHUNK2, NBUF2 = 128, 80, 4
E_PAD1 = NW * NCHUNK1 * CHUNK1
E_PAD2 = NW * NCHUNK2 * CHUNK2
RB = 640
NB = N_PAD // RB


def _sc_degree(dst_r):
    mesh = plsc.VectorSubcoreMesh(core_axis_name="c", subcore_axis_name="s")

    @functools.partial(
        pl.kernel,
        out_type=jax.ShapeDtypeStruct((NC, N_PAD), jnp.float32),
        mesh=mesh,
        scratch_types=[
            pltpu.VMEM((NCHUNK, CHUNK), jnp.int32),
            pltpu.VMEM((CHUNK,), jnp.float32),
            pltpu.VMEM((ROWS_PT,), jnp.float32),
            pltpu.VMEM_SHARED((N_PAD,), jnp.float32),
        ],
    )
    def deg_kernel(dst_hbm, out_hbm, idx_v, ones_v, stage_v, acc_sh):
        c = lax.axis_index("c")
        s = lax.axis_index("s")
        wid = s * NC + c

        def zfill(i, _):
            stage_v[pl.ds(i * 16, 16)] = jnp.zeros((16,), jnp.float32)
            return 0

        lax.fori_loop(0, ROWS_PT // 16, zfill, 0)

        def ofill(i, _):
            ones_v[pl.ds(i * 16, 16)] = jnp.ones((16,), jnp.float32)
            return 0

        lax.fori_loop(0, CHUNK // 16, ofill, 0)

        pltpu.sync_copy(stage_v, acc_sh.at[pl.ds(s * ROWS_PT, ROWS_PT)])
        pltpu.sync_copy(dst_hbm.at[wid], idx_v)
        plsc.subcore_barrier()

        def edge_body(j, _):
            pltpu.sync_copy(ones_v, acc_sh.at[idx_v.at[j]], add=True)
            return 0

        lax.fori_loop(0, NCHUNK, edge_body, 0)
        plsc.subcore_barrier()
        pltpu.sync_copy(acc_sh.at[pl.ds(s * ROWS_PT, ROWS_PT)], stage_v)
        pltpu.sync_copy(stage_v, out_hbm.at[c, pl.ds(s * ROWS_PT, ROWS_PT)])

    return deg_kernel(dst_r)


def _sc_spmm(p, src_r, dst_r, d, chunk, nchunk, nbuf):
    mesh = plsc.VectorSubcoreMesh(core_axis_name="c", subcore_axis_name="s")

    pieces = []
    off = 0
    while off < ROWS_PT:
        sz = min(chunk, ROWS_PT - off)
        pieces.append((off, sz))
        off += sz

    @functools.partial(
        pl.kernel,
        out_type=jax.ShapeDtypeStruct((NC, N_PAD, d), jnp.float32),
        mesh=mesh,
        scratch_types=[
            pltpu.VMEM((nchunk, chunk), jnp.int32),
            pltpu.VMEM((nchunk, chunk), jnp.int32),
            pltpu.VMEM((nbuf, chunk, d), jnp.float32),
            pltpu.VMEM_SHARED((N_PAD, d), jnp.float32),
            pltpu.SemaphoreType.DMA((nbuf,)),
        ],
        compiler_params=pltpu.CompilerParams(use_tc_tiling_on_sc=False),
    )
    def spmm_kernel(p_hbm, src_hbm, dst_hbm, out_hbm, sidx, didx, rows,
                    acc_sh, gsem):
        c = lax.axis_index("c")
        s = lax.axis_index("s")
        wid = s * NC + c

        def zrow(i, _):
            for q in range(d // 16):
                rows[0, i, pl.ds(q * 16, 16)] = jnp.zeros((16,), jnp.float32)
            return 0

        lax.fori_loop(0, chunk, zrow, 0)
        for off, sz in pieces:
            pltpu.sync_copy(
                rows.at[0, pl.ds(0, sz)],
                acc_sh.at[pl.ds(s * ROWS_PT + off, sz)])
        pltpu.sync_copy(src_hbm.at[wid], sidx)
        pltpu.sync_copy(dst_hbm.at[wid], didx)
        for b in range(nbuf):
            pltpu.async_copy(p_hbm.at[sidx.at[b]], rows.at[b], gsem.at[b])
        plsc.subcore_barrier()

        def edge_body(jj, _):
            for b in range(nbuf):
                j = jj * nbuf + b
                pltpu.make_async_copy(
                    p_hbm.at[sidx.at[j]], rows.at[b], gsem.at[b]).wait()
                pltpu.sync_copy(rows.at[b], acc_sh.at[didx.at[j]], add=True)

                @pl.when(j + nbuf < nchunk)
                def _():
                    pltpu.async_copy(
                        p_hbm.at[sidx.at[j + nbuf]], rows.at[b], gsem.at[b])

            return 0

        lax.fori_loop(0, nchunk // nbuf, edge_body, 0)
        plsc.subcore_barrier()
        for off, sz in pieces:
            pltpu.sync_copy(
                acc_sh.at[pl.ds(s * ROWS_PT + off, sz)],
                rows.at[0, pl.ds(0, sz)])
            pltpu.sync_copy(
                rows.at[0, pl.ds(0, sz)],
                out_hbm.at[c, pl.ds(s * ROWS_PT + off, sz)])

    return spmm_kernel(p, src_r, dst_r)


def _tc1(x_pad, W1, d0, d1):
    def body(x_ref, w_ref, d0_ref, d1_ref, p_ref, dis_ref):
        deg = d0_ref[...] + d1_ref[...] + 1.0
        dis = lax.rsqrt(deg)
        h = jnp.dot(x_ref[...], w_ref[...], preferred_element_type=jnp.float32)
        p_ref[...] = h * dis
        dis_ref[...] = dis

    return pl.pallas_call(
        body,
        grid=(NB,),
        in_specs=[
            pl.BlockSpec((RB, D_IN), lambda i: (i, 0)),
            pl.BlockSpec((D_IN, H1), lambda i: (0, 0)),
            pl.BlockSpec((RB, 1), lambda i: (i, 0)),
            pl.BlockSpec((RB, 1), lambda i: (i, 0)),
        ],
        out_specs=[
            pl.BlockSpec((RB, H1), lambda i: (i, 0)),
            pl.BlockSpec((RB, 1), lambda i: (i, 0)),
        ],
        out_shape=[
            jax.ShapeDtypeStruct((N_PAD, H1), jnp.float32),
            jax.ShapeDtypeStruct((N_PAD, 1), jnp.float32),
        ],
    )(x_pad, W1, d0, d1)


def _tc2(s1a, s1b, p1, dis, b1, W2):
    def body(a_ref, b_ref, p_ref, dis_ref, bias_ref, w_ref, o_ref):
        dis = dis_ref[...]
        t = jnp.maximum(
            dis * (a_ref[...] + b_ref[...] + p_ref[...]) + bias_ref[...], 0.0)
        o_ref[...] = dis * jnp.dot(
            t, w_ref[...], preferred_element_type=jnp.float32)

    return pl.pallas_call(
        body,
        grid=(NB,),
        in_specs=[
            pl.BlockSpec((RB, H1), lambda i: (i, 0)),
            pl.BlockSpec((RB, H1), lambda i: (i, 0)),
            pl.BlockSpec((RB, H1), lambda i: (i, 0)),
            pl.BlockSpec((RB, 1), lambda i: (i, 0)),
            pl.BlockSpec((1, H1), lambda i: (0, 0)),
            pl.BlockSpec((H1, H2), lambda i: (0, 0)),
        ],
        out_specs=pl.BlockSpec((RB, H2), lambda i: (i, 0)),
        out_shape=jax.ShapeDtypeStruct((N_PAD, H2), jnp.float32),
    )(s1a, s1b, p1, dis, b1, W2)


def _tc3(s2a, s2b, p2, dis, b2, batch_col, Wfc, bfc):
    def body(a_ref, b_ref, p_ref, dis_ref, bias_ref, bt_ref, wfc_ref, bfc_ref,
             o_ref, sums, cnts):
        i = pl.program_id(0)

        @pl.when(i == 0)
        def _():
            sums[...] = jnp.zeros_like(sums)
            cnts[...] = jnp.zeros_like(cnts)

        dis = dis_ref[...]
        t = jnp.maximum(
            dis * (a_ref[...] + b_ref[...] + p_ref[...]) + bias_ref[...], 0.0)
        gids = lax.broadcasted_iota(jnp.int32, (RB, N_GRAPHS), 1)
        onehot = (bt_ref[...] == gids).astype(jnp.float32)
        sums[...] += lax.dot_general(
            onehot, t, (((0,), (0,)), ((), ())),
            preferred_element_type=jnp.float32,
            precision=lax.Precision.HIGHEST)
        cnts[...] += lax.dot_general(
            onehot, jnp.ones((RB, 1), jnp.float32), (((0,), (0,)), ((), ())),
            preferred_element_type=jnp.float32,
            precision=lax.Precision.HIGHEST)

        @pl.when(i == NB - 1)
        def _():
            g = sums[...] / jnp.maximum(cnts[...], 1.0)
            logits = jnp.dot(
                g, wfc_ref[...], preferred_element_type=jnp.float32,
                precision=lax.Precision.HIGHEST) + bfc_ref[...]
            m = jnp.max(logits, axis=1, keepdims=True)
            e = jnp.exp(logits - m)
            o_ref[...] = e / jnp.sum(e, axis=1, keepdims=True)

    return pl.pallas_call(
        body,
        grid=(NB,),
        in_specs=[
            pl.BlockSpec((RB, H2), lambda i: (i, 0)),
            pl.BlockSpec((RB, H2), lambda i: (i, 0)),
            pl.BlockSpec((RB, H2), lambda i: (i, 0)),
            pl.BlockSpec((RB, 1), lambda i: (i, 0)),
            pl.BlockSpec((1, H2), lambda i: (0, 0)),
            pl.BlockSpec((RB, 1), lambda i: (i, 0)),
            pl.BlockSpec((H2, D_OUT), lambda i: (0, 0)),
            pl.BlockSpec((1, D_OUT), lambda i: (0, 0)),
        ],
        out_specs=pl.BlockSpec((N_GRAPHS, D_OUT), lambda i: (0, 0)),
        out_shape=jax.ShapeDtypeStruct((N_GRAPHS, D_OUT), jnp.float32),
        scratch_shapes=[
            pltpu.VMEM((N_GRAPHS, H2), jnp.float32),
            pltpu.VMEM((N_GRAPHS, 1), jnp.float32),
        ],
    )(s2a, s2b, p2, dis, b2, batch_col, Wfc, bfc)


def kernel(x, edge_index, batch, W1, b1, W2, b2, Wfc, bfc):
    src = edge_index[0].astype(jnp.int32)
    dst = edge_index[1].astype(jnp.int32)
    n_edges = src.shape[0]

    def _edges(arr, e_pad, nchunk, chunk):
        fill = jnp.full((e_pad - n_edges,), DUMMY, dtype=jnp.int32)
        return jnp.concatenate([arr, fill]).reshape(NW, nchunk, chunk)

    src_r1 = _edges(src, E_PAD1, NCHUNK1, CHUNK1)
    dst_r1 = _edges(dst, E_PAD1, NCHUNK1, CHUNK1)
    src_r2 = _edges(src, E_PAD2, NCHUNK2, CHUNK2)
    dst_r2 = _edges(dst, E_PAD2, NCHUNK2, CHUNK2)
    dst_r = _edges(dst, E_PAD, NCHUNK, CHUNK)
    x_pad = jnp.pad(x, ((0, N_PAD - N_NODES), (0, 0)))
    batch_col = jnp.pad(
        batch.astype(jnp.int32), (0, N_PAD - N_NODES),
        constant_values=N_GRAPHS).reshape(N_PAD, 1)

    deg_parts = _sc_degree(dst_r)
    d0 = deg_parts[0].reshape(N_PAD, 1)
    d1 = deg_parts[1].reshape(N_PAD, 1)

    p1, dis = _tc1(x_pad, W1, d0, d1)
    s1 = _sc_spmm(p1, src_r1, dst_r1, H1, CHUNK1, NCHUNK1, NBUF1)
    p2 = _tc2(s1[0], s1[1], p1, dis, b1.reshape(1, H1), W2)
    s2 = _sc_spmm(p2, src_r2, dst_r2, H2, CHUNK2, NCHUNK2, NBUF2)
    return _tc3(s2[0], s2[1], p2, dis, b2.reshape(1, H2), batch_col,
                Wfc, bfc.reshape(1, D_OUT))


# speedup vs baseline: 34.4648x; 1.9651x over previous
import functools

import jax
import jax.numpy as jnp
from jax import lax
from jax.experimental import pallas as pl
from jax.experimental.pallas import tpu as pltpu
from jax.experimental.pallas import tpu_sc as plsc

N_NODES = 10000
N_GRAPHS = 64
D_IN = 128
H1 = 128
H2 = 64
H2P = 128
D_OUT = 10

NC = 2
NS = 16
NW = NC * NS
CHUNK = 128
N_PAD = 10240
DUMMY = 10016
ROWS_PT = N_PAD // NS
NCHUNK = 80
E_PAD = NW * NCHUNK * CHUNK
CHUNK1, NCHUNK1, NBUF1 = 112, 90, 2
CHUNK2, NCHUNK2, NBUF2 = 128, 80, 4
E_PAD1 = NW * NCHUNK1 * CHUNK1
E_PAD2 = NW * NCHUNK2 * CHUNK2
RB = 640
NB = N_PAD // RB


def _sc_degree(dst_r):
    mesh = plsc.VectorSubcoreMesh(core_axis_name="c", subcore_axis_name="s")

    @functools.partial(
        pl.kernel,
        out_type=jax.ShapeDtypeStruct((NC, N_PAD), jnp.float32),
        mesh=mesh,
        scratch_types=[
            pltpu.VMEM((NCHUNK, CHUNK), jnp.int32),
            pltpu.VMEM((CHUNK,), jnp.float32),
            pltpu.VMEM((ROWS_PT,), jnp.float32),
            pltpu.VMEM_SHARED((N_PAD,), jnp.float32),
        ],
    )
    def deg_kernel(dst_hbm, out_hbm, idx_v, ones_v, stage_v, acc_sh):
        c = lax.axis_index("c")
        s = lax.axis_index("s")
        wid = s * NC + c

        def zfill(i, _):
            stage_v[pl.ds(i * 16, 16)] = jnp.zeros((16,), jnp.float32)
            return 0

        lax.fori_loop(0, ROWS_PT // 16, zfill, 0)

        def ofill(i, _):
            ones_v[pl.ds(i * 16, 16)] = jnp.ones((16,), jnp.float32)
            return 0

        lax.fori_loop(0, CHUNK // 16, ofill, 0)

        pltpu.sync_copy(stage_v, acc_sh.at[pl.ds(s * ROWS_PT, ROWS_PT)])
        pltpu.sync_copy(dst_hbm.at[wid], idx_v)
        plsc.subcore_barrier()

        def edge_body(j, _):
            pltpu.sync_copy(ones_v, acc_sh.at[idx_v.at[j]], add=True)
            return 0

        lax.fori_loop(0, NCHUNK, edge_body, 0)
        plsc.subcore_barrier()
        pltpu.sync_copy(acc_sh.at[pl.ds(s * ROWS_PT, ROWS_PT)], stage_v)
        pltpu.sync_copy(stage_v, out_hbm.at[c, pl.ds(s * ROWS_PT, ROWS_PT)])

    return deg_kernel(dst_r)


def _sc_spmm(p, src_r, dst_r, d, chunk, nchunk, nbuf):
    mesh = plsc.VectorSubcoreMesh(core_axis_name="c", subcore_axis_name="s")

    pieces = []
    off = 0
    while off < ROWS_PT:
        sz = min(chunk, ROWS_PT - off)
        pieces.append((off, sz))
        off += sz

    @functools.partial(
        pl.kernel,
        out_type=jax.ShapeDtypeStruct((NC, N_PAD, d), jnp.float32),
        mesh=mesh,
        scratch_types=[
            pltpu.VMEM((nchunk, chunk), jnp.int32),
            pltpu.VMEM((nchunk, chunk), jnp.int32),
            pltpu.VMEM((nbuf, chunk, d), jnp.float32),
            pltpu.VMEM_SHARED((N_PAD, d), jnp.float32),
            pltpu.SemaphoreType.DMA((nbuf,)),
        ],
        compiler_params=pltpu.CompilerParams(use_tc_tiling_on_sc=False),
    )
    def spmm_kernel(p_hbm, src_hbm, dst_hbm, out_hbm, sidx, didx, rows,
                    acc_sh, gsem):
        c = lax.axis_index("c")
        s = lax.axis_index("s")
        wid = s * NC + c

        def zrow(i, _):
            for q in range(d // 16):
                rows[0, i, pl.ds(q * 16, 16)] = jnp.zeros((16,), jnp.float32)
            return 0

        lax.fori_loop(0, chunk, zrow, 0)
        for off, sz in pieces:
            pltpu.sync_copy(
                rows.at[0, pl.ds(0, sz)],
                acc_sh.at[pl.ds(s * ROWS_PT + off, sz)])
        pltpu.sync_copy(src_hbm.at[wid], sidx)
        pltpu.sync_copy(dst_hbm.at[wid], didx)
        for b in range(nbuf):
            pltpu.async_copy(p_hbm.at[sidx.at[b]], rows.at[b], gsem.at[b])
        plsc.subcore_barrier()

        def edge_body(jj, _):
            for b in range(nbuf):
                j = jj * nbuf + b
                pltpu.make_async_copy(
                    p_hbm.at[sidx.at[j]], rows.at[b], gsem.at[b]).wait()
                pltpu.sync_copy(rows.at[b], acc_sh.at[didx.at[j]], add=True)

                @pl.when(j + nbuf < nchunk)
                def _():
                    pltpu.async_copy(
                        p_hbm.at[sidx.at[j + nbuf]], rows.at[b], gsem.at[b])

            return 0

        lax.fori_loop(0, nchunk // nbuf, edge_body, 0)
        plsc.subcore_barrier()
        for off, sz in pieces:
            pltpu.sync_copy(
                acc_sh.at[pl.ds(s * ROWS_PT + off, sz)],
                rows.at[0, pl.ds(0, sz)])
            pltpu.sync_copy(
                rows.at[0, pl.ds(0, sz)],
                out_hbm.at[c, pl.ds(s * ROWS_PT + off, sz)])

    return spmm_kernel(p, src_r, dst_r)


def _tc1(x_pad, W1, d0, d1):
    def body(x_ref, w_ref, d0_ref, d1_ref, p_ref, dis_ref):
        deg = d0_ref[...] + d1_ref[...] + 1.0
        dis = lax.rsqrt(deg)
        h = jnp.dot(x_ref[...], w_ref[...], preferred_element_type=jnp.float32)
        p_ref[...] = h * dis
        dis_ref[...] = dis

    return pl.pallas_call(
        body,
        grid=(NB,),
        in_specs=[
            pl.BlockSpec((RB, D_IN), lambda i: (i, 0)),
            pl.BlockSpec((D_IN, H1), lambda i: (0, 0)),
            pl.BlockSpec((RB, 1), lambda i: (i, 0)),
            pl.BlockSpec((RB, 1), lambda i: (i, 0)),
        ],
        out_specs=[
            pl.BlockSpec((RB, H1), lambda i: (i, 0)),
            pl.BlockSpec((RB, 1), lambda i: (i, 0)),
        ],
        out_shape=[
            jax.ShapeDtypeStruct((N_PAD, H1), jnp.float32),
            jax.ShapeDtypeStruct((N_PAD, 1), jnp.float32),
        ],
    )(x_pad, W1, d0, d1)


def _tc2(s1a, s1b, p1, dis, b1, W2):
    def body(a_ref, b_ref, p_ref, dis_ref, bias_ref, w_ref, o_ref):
        dis = dis_ref[...]
        t = jnp.maximum(
            dis * (a_ref[...] + b_ref[...] + p_ref[...]) + bias_ref[...], 0.0)
        o_ref[...] = dis * jnp.dot(
            t, w_ref[...], preferred_element_type=jnp.float32)

    return pl.pallas_call(
        body,
        grid=(NB,),
        in_specs=[
            pl.BlockSpec((RB, H1), lambda i: (i, 0)),
            pl.BlockSpec((RB, H1), lambda i: (i, 0)),
            pl.BlockSpec((RB, H1), lambda i: (i, 0)),
            pl.BlockSpec((RB, 1), lambda i: (i, 0)),
            pl.BlockSpec((1, H1), lambda i: (0, 0)),
            pl.BlockSpec((H1, H2), lambda i: (0, 0)),
        ],
        out_specs=pl.BlockSpec((RB, H2), lambda i: (i, 0)),
        out_shape=jax.ShapeDtypeStruct((N_PAD, H2), jnp.float32),
    )(s1a, s1b, p1, dis, b1, W2)


def _tc3(s2a, s2b, p2, dis, b2, batch_col, Wfc, bfc):
    def body(a_ref, b_ref, p_ref, dis_ref, bias_ref, bt_ref, wfc_ref, bfc_ref,
             o_ref, sums, cnts):
        i = pl.program_id(0)

        @pl.when(i == 0)
        def _():
            sums[...] = jnp.zeros_like(sums)
            cnts[...] = jnp.zeros_like(cnts)

        dis = dis_ref[...]
        t = jnp.maximum(
            dis * (a_ref[...] + b_ref[...] + p_ref[...]) + bias_ref[...], 0.0)
        gids = lax.broadcasted_iota(jnp.int32, (RB, N_GRAPHS), 1)
        onehot = (bt_ref[...] == gids).astype(jnp.float32)
        sums[...] += lax.dot_general(
            onehot, t, (((0,), (0,)), ((), ())),
            preferred_element_type=jnp.float32,
            precision=lax.Precision.HIGHEST)
        cnts[...] += lax.dot_general(
            onehot, jnp.ones((RB, 1), jnp.float32), (((0,), (0,)), ((), ())),
            preferred_element_type=jnp.float32,
            precision=lax.Precision.HIGHEST)

        @pl.when(i == NB - 1)
        def _():
            g = sums[...] / jnp.maximum(cnts[...], 1.0)
            logits = jnp.dot(
                g, wfc_ref[...], preferred_element_type=jnp.float32,
                precision=lax.Precision.HIGHEST) + bfc_ref[...]
            m = jnp.max(logits, axis=1, keepdims=True)
            e = jnp.exp(logits - m)
            o_ref[...] = e / jnp.sum(e, axis=1, keepdims=True)

    return pl.pallas_call(
        body,
        grid=(NB,),
        in_specs=[
            pl.BlockSpec((RB, H2), lambda i: (i, 0)),
            pl.BlockSpec((RB, H2), lambda i: (i, 0)),
            pl.BlockSpec((RB, H2), lambda i: (i, 0)),
            pl.BlockSpec((RB, 1), lambda i: (i, 0)),
            pl.BlockSpec((1, H2), lambda i: (0, 0)),
            pl.BlockSpec((RB, 1), lambda i: (i, 0)),
            pl.BlockSpec((H2, D_OUT), lambda i: (0, 0)),
            pl.BlockSpec((1, D_OUT), lambda i: (0, 0)),
        ],
        out_specs=pl.BlockSpec((N_GRAPHS, D_OUT), lambda i: (0, 0)),
        out_shape=jax.ShapeDtypeStruct((N_GRAPHS, D_OUT), jnp.float32),
        scratch_shapes=[
            pltpu.VMEM((N_GRAPHS, H2), jnp.float32),
            pltpu.VMEM((N_GRAPHS, 1), jnp.float32),
        ],
    )(s2a, s2b, p2, dis, b2, batch_col, Wfc, bfc)


def kernel(x, edge_index, batch, W1, b1, W2, b2, Wfc, bfc):
    src = edge_index[0].astype(jnp.int32)
    dst = edge_index[1].astype(jnp.int32)
    n_edges = src.shape[0]

    def _edges(arr, e_pad, nchunk, chunk):
        fill = DUMMY + jnp.arange(e_pad - n_edges, dtype=jnp.int32) % (
            N_PAD - DUMMY)
        return jnp.concatenate([arr, fill]).reshape(NW, nchunk, chunk)

    src_r1 = _edges(src, E_PAD1, NCHUNK1, CHUNK1)
    dst_r1 = _edges(dst, E_PAD1, NCHUNK1, CHUNK1)
    src_r2 = _edges(src, E_PAD2, NCHUNK2, CHUNK2)
    dst_r2 = _edges(dst, E_PAD2, NCHUNK2, CHUNK2)
    dst_r = _edges(dst, E_PAD, NCHUNK, CHUNK)
    x_pad = jnp.pad(x, ((0, N_PAD - N_NODES), (0, 0)))
    batch_col = jnp.pad(
        batch.astype(jnp.int32), (0, N_PAD - N_NODES),
        constant_values=N_GRAPHS).reshape(N_PAD, 1)

    deg_parts = _sc_degree(dst_r)
    d0 = deg_parts[0].reshape(N_PAD, 1)
    d1 = deg_parts[1].reshape(N_PAD, 1)

    p1, dis = _tc1(x_pad, W1, d0, d1)
    s1 = _sc_spmm(p1, src_r1, dst_r1, H1, CHUNK1, NCHUNK1, NBUF1)
    p2 = _tc2(s1[0], s1[1], p1, dis, b1.reshape(1, H1), W2)
    s2 = _sc_spmm(p2, src_r2, dst_r2, H2, CHUNK2, NCHUNK2, NBUF2)
    return _tc3(s2[0], s2[1], p2, dis, b2.reshape(1, H2), batch_col,
                Wfc, bfc.reshape(1, D_OUT))


# speedup vs baseline: 39.4021x; 1.1433x over previous
import functools

import jax
import jax.numpy as jnp
from jax import lax
from jax.experimental import pallas as pl
from jax.experimental.pallas import tpu as pltpu
from jax.experimental.pallas import tpu_sc as plsc

N_NODES = 10000
N_GRAPHS = 64
D_IN = 128
H1 = 128
H2 = 64
D_OUT = 10

NC = 2
NS = 16
NW = NC * NS
CHUNK = 128
NCHUNK = 80
HC = NCHUNK // 2
E_PAD = NW * NCHUNK * CHUNK
N_PAD = 10240
DUMMY = 10016
ROWS_PT = N_PAD // NS
NBUF1 = 2
NBUF2 = 4
RB = 1280
NB = N_PAD // RB


def _sc_degree(dst_r):
    mesh = plsc.VectorSubcoreMesh(core_axis_name="c", subcore_axis_name="s")

    @functools.partial(
        pl.kernel,
        out_type=jax.ShapeDtypeStruct((NC, N_PAD), jnp.float32),
        mesh=mesh,
        scratch_types=[
            pltpu.VMEM((NCHUNK, CHUNK), jnp.int32),
            pltpu.VMEM((CHUNK,), jnp.float32),
            pltpu.VMEM((ROWS_PT,), jnp.float32),
            pltpu.VMEM_SHARED((N_PAD,), jnp.float32),
        ],
    )
    def deg_kernel(dst_hbm, out_hbm, idx_v, ones_v, stage_v, acc_sh):
        c = lax.axis_index("c")
        s = lax.axis_index("s")
        wid = s * NC + c

        def zfill(i, _):
            stage_v[pl.ds(i * 16, 16)] = jnp.zeros((16,), jnp.float32)
            return 0

        lax.fori_loop(0, ROWS_PT // 16, zfill, 0)

        def ofill(i, _):
            ones_v[pl.ds(i * 16, 16)] = jnp.ones((16,), jnp.float32)
            return 0

        lax.fori_loop(0, CHUNK // 16, ofill, 0)

        pltpu.sync_copy(stage_v, acc_sh.at[pl.ds(s * ROWS_PT, ROWS_PT)])
        pltpu.sync_copy(dst_hbm.at[wid], idx_v)
        plsc.subcore_barrier()

        def edge_body(j, _):
            pltpu.sync_copy(ones_v, acc_sh.at[idx_v.at[j]], add=True)
            return 0

        lax.fori_loop(0, NCHUNK, edge_body, 0)
        plsc.subcore_barrier()
        pltpu.sync_copy(acc_sh.at[pl.ds(s * ROWS_PT, ROWS_PT)], stage_v)
        pltpu.sync_copy(stage_v, out_hbm.at[c, pl.ds(s * ROWS_PT, ROWS_PT)])

    return deg_kernel(dst_r)


def _sc_spmm(p, src_r, dst_r, d, nbuf):
    mesh = plsc.VectorSubcoreMesh(core_axis_name="c", subcore_axis_name="s")

    @functools.partial(
        pl.kernel,
        out_type=jax.ShapeDtypeStruct((NC, N_PAD, d), jnp.float32),
        mesh=mesh,
        scratch_types=[
            pltpu.VMEM((NCHUNK, CHUNK), jnp.int32),
            pltpu.VMEM((HC, CHUNK), jnp.int32),
            pltpu.VMEM((nbuf, CHUNK, d), jnp.float32),
            pltpu.VMEM_SHARED((N_PAD, d), jnp.float32),
            pltpu.SemaphoreType.DMA((nbuf,)),
        ],
        compiler_params=pltpu.CompilerParams(use_tc_tiling_on_sc=False),
    )
    def spmm_kernel(p_hbm, src_hbm, dst_hbm, out_hbm, sidx, didx, rows,
                    acc_sh, gsem):
        c = lax.axis_index("c")
        s = lax.axis_index("s")
        wid = s * NC + c

        def zrow(i, _):
            for q in range(d // 16):
                rows[0, i, pl.ds(q * 16, 16)] = jnp.zeros((16,), jnp.float32)
            return 0

        lax.fori_loop(0, CHUNK, zrow, 0)
        for t in range(ROWS_PT // CHUNK):
            pltpu.sync_copy(
                rows.at[0], acc_sh.at[pl.ds(s * ROWS_PT + t * CHUNK, CHUNK)])
        pltpu.sync_copy(src_hbm.at[wid], sidx)
        for b in range(nbuf):
            pltpu.async_copy(p_hbm.at[sidx.at[b]], rows.at[b], gsem.at[b])
        plsc.subcore_barrier()

        for h in range(2):
            pltpu.sync_copy(dst_hbm.at[wid, pl.ds(h * HC, HC)], didx)

            def edge_body(jj, _, h=h):
                for b in range(nbuf):
                    jloc = jj * nbuf + b
                    j = h * HC + jloc
                    pltpu.make_async_copy(
                        p_hbm.at[sidx.at[j]], rows.at[b], gsem.at[b]).wait()
                    pltpu.sync_copy(
                        rows.at[b], acc_sh.at[didx.at[jloc]], add=True)

                    @pl.when(j + nbuf < NCHUNK)
                    def _():
                        pltpu.async_copy(
                            p_hbm.at[sidx.at[j + nbuf]], rows.at[b],
                            gsem.at[b])

                return 0

            lax.fori_loop(0, HC // nbuf, edge_body, 0)

        plsc.subcore_barrier()
        for t in range(ROWS_PT // CHUNK):
            pltpu.sync_copy(
                acc_sh.at[pl.ds(s * ROWS_PT + t * CHUNK, CHUNK)], rows.at[0])
            pltpu.sync_copy(
                rows.at[0], out_hbm.at[c, pl.ds(s * ROWS_PT + t * CHUNK, CHUNK)])

    return spmm_kernel(p, src_r, dst_r)


def _tc_h1(x, W1):
    def body(x_ref, w_ref, h_ref):
        i = pl.program_id(0)
        h = jnp.dot(x_ref[...], w_ref[...], preferred_element_type=jnp.float32)
        rows = lax.broadcasted_iota(jnp.int32, (RB, 1), 0) + i * RB
        h_ref[...] = jnp.where(rows < N_NODES, h, 0.0)

    return pl.pallas_call(
        body,
        grid=(NB,),
        in_specs=[
            pl.BlockSpec((RB, D_IN), lambda i: (i, 0)),
            pl.BlockSpec((D_IN, H1), lambda i: (0, 0)),
        ],
        out_specs=pl.BlockSpec((RB, H1), lambda i: (i, 0)),
        out_shape=jax.ShapeDtypeStruct((N_PAD, H1), jnp.float32),
    )(x, W1)


def _tc_scale(h1, deg_parts):
    def body(h_ref, dp_ref, p_ref, dis_ref):
        deg = dp_ref[0] + dp_ref[1] + 1.0
        dis = lax.rsqrt(deg)
        p_ref[...] = h_ref[...] * dis
        dis_ref[...] = dis

    return pl.pallas_call(
        body,
        grid=(NB,),
        in_specs=[
            pl.BlockSpec((RB, H1), lambda i: (i, 0)),
            pl.BlockSpec((NC, RB, 1), lambda i: (0, i, 0)),
        ],
        out_specs=[
            pl.BlockSpec((RB, H1), lambda i: (i, 0)),
            pl.BlockSpec((RB, 1), lambda i: (i, 0)),
        ],
        out_shape=[
            jax.ShapeDtypeStruct((N_PAD, H1), jnp.float32),
            jax.ShapeDtypeStruct((N_PAD, 1), jnp.float32),
        ],
    )(h1, deg_parts)


def _tc2(s1, p1, dis, b1, W2):
    def body(s_ref, p_ref, dis_ref, bias_ref, w_ref, o_ref):
        dis = dis_ref[...]
        t = jnp.maximum(
            dis * (s_ref[0] + s_ref[1] + p_ref[...]) + bias_ref[...], 0.0)
        o_ref[...] = dis * jnp.dot(
            t, w_ref[...], preferred_element_type=jnp.float32)

    return pl.pallas_call(
        body,
        grid=(NB,),
        in_specs=[
            pl.BlockSpec((NC, RB, H1), lambda i: (0, i, 0)),
            pl.BlockSpec((RB, H1), lambda i: (i, 0)),
            pl.BlockSpec((RB, 1), lambda i: (i, 0)),
            pl.BlockSpec((1, H1), lambda i: (0, 0)),
            pl.BlockSpec((H1, H2), lambda i: (0, 0)),
        ],
        out_specs=pl.BlockSpec((RB, H2), lambda i: (i, 0)),
        out_shape=jax.ShapeDtypeStruct((N_PAD, H2), jnp.float32),
    )(s1, p1, dis, b1, W2)


def _tc3(s2, p2, dis, b2, batch_col, Wfc, bfc):
    def body(s_ref, p_ref, dis_ref, bias_ref, bt_ref, wfc_ref, bfc_ref,
             o_ref, sums, cnts):
        i = pl.program_id(0)

        @pl.when(i == 0)
        def _():
            sums[...] = jnp.zeros_like(sums)
            cnts[...] = jnp.zeros_like(cnts)

        dis = dis_ref[...]
        t = jnp.maximum(
            dis * (s_ref[0] + s_ref[1] + p_ref[...]) + bias_ref[...], 0.0)
        gids = lax.broadcasted_iota(jnp.int32, (RB, N_GRAPHS), 1)
        rows = lax.broadcasted_iota(jnp.int32, (RB, 1), 0) + i * RB
        onehot = ((bt_ref[...] == gids) & (rows < N_NODES)).astype(jnp.float32)
        sums[...] += lax.dot_general(
            onehot, t, (((0,), (0,)), ((), ())),
            preferred_element_type=jnp.float32,
            precision=lax.Precision.HIGHEST)
        cnts[...] += lax.dot_general(
            onehot, jnp.ones((RB, 1), jnp.float32), (((0,), (0,)), ((), ())),
            preferred_element_type=jnp.float32,
            precision=lax.Precision.HIGHEST)

        @pl.when(i == NB - 1)
        def _():
            g = sums[...] / jnp.maximum(cnts[...], 1.0)
            logits = jnp.dot(
                g, wfc_ref[...], preferred_element_type=jnp.float32,
                precision=lax.Precision.HIGHEST) + bfc_ref[...]
            m = jnp.max(logits, axis=1, keepdims=True)
            e = jnp.exp(logits - m)
            o_ref[...] = e / jnp.sum(e, axis=1, keepdims=True)

    return pl.pallas_call(
        body,
        grid=(NB,),
        in_specs=[
            pl.BlockSpec((NC, RB, H2), lambda i: (0, i, 0)),
            pl.BlockSpec((RB, H2), lambda i: (i, 0)),
            pl.BlockSpec((RB, 1), lambda i: (i, 0)),
            pl.BlockSpec((1, H2), lambda i: (0, 0)),
            pl.BlockSpec((RB, 1), lambda i: (i, 0)),
            pl.BlockSpec((H2, D_OUT), lambda i: (0, 0)),
            pl.BlockSpec((1, D_OUT), lambda i: (0, 0)),
        ],
        out_specs=pl.BlockSpec((N_GRAPHS, D_OUT), lambda i: (0, 0)),
        out_shape=jax.ShapeDtypeStruct((N_GRAPHS, D_OUT), jnp.float32),
        scratch_shapes=[
            pltpu.VMEM((N_GRAPHS, H2), jnp.float32),
            pltpu.VMEM((N_GRAPHS, 1), jnp.float32),
        ],
    )(s2, p2, dis, b2, batch_col, Wfc, bfc)


def kernel(x, edge_index, batch, W1, b1, W2, b2, Wfc, bfc):
    n_edges = edge_index.shape[1]
    fill = DUMMY + jnp.arange(E_PAD - n_edges, dtype=jnp.int32) % (
        N_PAD - DUMMY)
    edges = jnp.concatenate(
        [edge_index.astype(jnp.int32),
         jnp.broadcast_to(fill, (2, E_PAD - n_edges))], axis=1)
    src_r = edges[0].reshape(NW, NCHUNK, CHUNK)
    dst_r = edges[1].reshape(NW, NCHUNK, CHUNK)
    batch_col = batch.astype(jnp.int32).reshape(N_NODES, 1)

    deg_parts = _sc_degree(dst_r)
    h1 = _tc_h1(x, W1)
    p1, dis = _tc_scale(h1, deg_parts.reshape(NC, N_PAD, 1))
    s1 = _sc_spmm(p1, src_r, dst_r, H1, NBUF1)
    p2 = _tc2(s1, p1, dis, b1.reshape(1, H1), W2)
    s2 = _sc_spmm(p2, src_r, dst_r, H2, NBUF2)
    return _tc3(s2, p2, dis, b2.reshape(1, H2), batch_col,
                Wfc, bfc.reshape(1, D_OUT))


# speedup vs baseline: 41.5187x; 1.0537x over previous
import functools

import jax
import jax.numpy as jnp
from jax import lax
from jax.experimental import pallas as pl
from jax.experimental.pallas import tpu as pltpu
from jax.experimental.pallas import tpu_sc as plsc

N_NODES = 10000
N_GRAPHS = 64
D_IN = 128
H1 = 128
H2 = 64
D_OUT = 10
N_EDGES = 320000

NC = 2
NS = 16
NW = NC * NS
CHUNK = 128
NCH_TOT = N_EDGES // CHUNK
BASE_CH = NCH_TOT // NW
EXTRA = NCH_TOT % NW
MAX_CH = BASE_CH + 1
HALF = 40
N_PAD = 10240
ROWS_PT = N_PAD // NS
NBUF1 = 2
NBUF2 = 4
RB = 1280
NB = N_PAD // RB


def _worker_range(wid):
    start = wid * BASE_CH + jnp.minimum(wid, EXTRA)
    nch = BASE_CH + jnp.where(wid < EXTRA, 1, 0)
    return start, nch


def _load_chunks(src_hbm, start, wid, buf):
    pltpu.sync_copy(src_hbm.at[pl.ds(start, BASE_CH)],
                    buf.at[pl.ds(0, BASE_CH)])

    @pl.when(wid < EXTRA)
    def _():
        pltpu.sync_copy(src_hbm.at[pl.ds(start + BASE_CH, 1)],
                        buf.at[pl.ds(BASE_CH, 1)])


def _sc_degree(dst_r):
    mesh = plsc.VectorSubcoreMesh(core_axis_name="c", subcore_axis_name="s")

    @functools.partial(
        pl.kernel,
        out_type=jax.ShapeDtypeStruct((NC, N_PAD), jnp.float32),
        mesh=mesh,
        scratch_types=[
            pltpu.VMEM((MAX_CH, CHUNK), jnp.int32),
            pltpu.VMEM((CHUNK,), jnp.float32),
            pltpu.VMEM((ROWS_PT,), jnp.float32),
            pltpu.VMEM_SHARED((N_PAD,), jnp.float32),
        ],
        compiler_params=pltpu.CompilerParams(use_tc_tiling_on_sc=False),
    )
    def deg_kernel(dst_hbm, out_hbm, idx_v, ones_v, stage_v, acc_sh):
        c = lax.axis_index("c")
        s = lax.axis_index("s")
        wid = s * NC + c
        start, nch = _worker_range(wid)

        def zfill(i, _):
            stage_v[pl.ds(i * 16, 16)] = jnp.zeros((16,), jnp.float32)
            return 0

        lax.fori_loop(0, ROWS_PT // 16, zfill, 0)

        def ofill(i, _):
            ones_v[pl.ds(i * 16, 16)] = jnp.ones((16,), jnp.float32)
            return 0

        lax.fori_loop(0, CHUNK // 16, ofill, 0)

        pltpu.sync_copy(stage_v, acc_sh.at[pl.ds(s * ROWS_PT, ROWS_PT)])
        _load_chunks(dst_hbm, start, wid, idx_v)
        plsc.subcore_barrier()

        def edge_body(j, _):
            pltpu.sync_copy(ones_v, acc_sh.at[idx_v.at[j]], add=True)
            return 0

        lax.fori_loop(0, nch, edge_body, 0)
        plsc.subcore_barrier()
        pltpu.sync_copy(acc_sh.at[pl.ds(s * ROWS_PT, ROWS_PT)], stage_v)
        pltpu.sync_copy(stage_v, out_hbm.at[c, pl.ds(s * ROWS_PT, ROWS_PT)])

    return deg_kernel(dst_r)


def _sc_spmm(p, src_r, dst_r, d, nbuf, out_width):
    mesh = plsc.VectorSubcoreMesh(core_axis_name="c", subcore_axis_name="s")

    @functools.partial(
        pl.kernel,
        out_type=jax.ShapeDtypeStruct((NC, N_PAD, out_width), jnp.float32),
        mesh=mesh,
        scratch_types=[
            pltpu.VMEM((MAX_CH, CHUNK), jnp.int32),
            pltpu.VMEM((HALF, CHUNK), jnp.int32),
            pltpu.VMEM((nbuf, CHUNK, d), jnp.float32),
            pltpu.VMEM_SHARED((N_PAD, d), jnp.float32),
            pltpu.SemaphoreType.DMA((nbuf,)),
        ],
        compiler_params=pltpu.CompilerParams(use_tc_tiling_on_sc=False),
    )
    def spmm_kernel(p_hbm, src_hbm, dst_hbm, out_hbm, sidx, didx, rows,
                    acc_sh, gsem):
        c = lax.axis_index("c")
        s = lax.axis_index("s")
        wid = s * NC + c
        start, nch = _worker_range(wid)

        def zrow(i, _):
            for q in range(d // 16):
                rows[0, i, pl.ds(q * 16, 16)] = jnp.zeros((16,), jnp.float32)
            return 0

        lax.fori_loop(0, CHUNK, zrow, 0)
        for t in range(ROWS_PT // CHUNK):
            pltpu.sync_copy(
                rows.at[0], acc_sh.at[pl.ds(s * ROWS_PT + t * CHUNK, CHUNK)])
        _load_chunks(src_hbm, start, wid, sidx)
        for b in range(nbuf):
            pltpu.async_copy(p_hbm.at[sidx.at[b]], rows.at[b], gsem.at[b])
        plsc.subcore_barrier()

        def gather_wait(j, b):
            pltpu.make_async_copy(
                p_hbm.at[sidx.at[j]], rows.at[b], gsem.at[b]).wait()

        def issue_ahead(j, b):
            @pl.when(j + nbuf < nch)
            def _():
                pltpu.async_copy(
                    p_hbm.at[sidx.at[j + nbuf]], rows.at[b], gsem.at[b])

        pltpu.sync_copy(dst_hbm.at[pl.ds(start, HALF)], didx)

        def body0(jj, _):
            for b in range(nbuf):
                j = jj * nbuf + b
                gather_wait(j, b)
                pltpu.sync_copy(rows.at[b], acc_sh.at[didx.at[j]], add=True)
                issue_ahead(j, b)
            return 0

        lax.fori_loop(0, HALF // nbuf, body0, 0)

        pltpu.sync_copy(dst_hbm.at[pl.ds(start + nch - HALF, HALF)], didx)

        def body1(j, _):
            b = lax.rem(j, nbuf)
            gather_wait(j, b)
            pltpu.sync_copy(
                rows.at[b], acc_sh.at[didx.at[j - (nch - HALF)]], add=True)
            issue_ahead(j, b)
            return 0

        lax.fori_loop(HALF, nch, body1, 0)

        plsc.subcore_barrier()
        for t in range(ROWS_PT // CHUNK):
            r0 = s * ROWS_PT + t * CHUNK
            pltpu.sync_copy(acc_sh.at[pl.ds(r0, CHUNK)], rows.at[0])
            if out_width == d:
                pltpu.sync_copy(rows.at[0], out_hbm.at[c, pl.ds(r0, CHUNK)])
            else:
                pltpu.sync_copy(
                    rows.at[0], out_hbm.at[c, pl.ds(r0, CHUNK), pl.ds(0, d)])

    return spmm_kernel(p, src_r, dst_r)


def _tc_h1(x, W1):
    def body(x_ref, w_ref, h_ref):
        i = pl.program_id(0)
        h = jnp.dot(x_ref[...], w_ref[...], preferred_element_type=jnp.float32)
        rows = lax.broadcasted_iota(jnp.int32, (RB, 1), 0) + i * RB
        h_ref[...] = jnp.where(rows < N_NODES, h, 0.0)

    return pl.pallas_call(
        body,
        grid=(NB,),
        in_specs=[
            pl.BlockSpec((RB, D_IN), lambda i: (i, 0)),
            pl.BlockSpec((D_IN, H1), lambda i: (0, 0)),
        ],
        out_specs=pl.BlockSpec((RB, H1), lambda i: (i, 0)),
        out_shape=jax.ShapeDtypeStruct((N_PAD, H1), jnp.float32),
    )(x, W1)


def _tc_scale(h1, deg_parts):
    def body(h_ref, dp_ref, p_ref, dis_ref):
        deg = dp_ref[0] + dp_ref[1] + 1.0
        dis = jnp.reshape(lax.rsqrt(deg), (RB, 1))
        p_ref[...] = h_ref[...] * dis
        dis_ref[...] = dis

    return pl.pallas_call(
        body,
        grid=(NB,),
        in_specs=[
            pl.BlockSpec((RB, H1), lambda i: (i, 0)),
            pl.BlockSpec((NC, RB), lambda i: (0, i)),
        ],
        out_specs=[
            pl.BlockSpec((RB, H1), lambda i: (i, 0)),
            pl.BlockSpec((RB, 1), lambda i: (i, 0)),
        ],
        out_shape=[
            jax.ShapeDtypeStruct((N_PAD, H1), jnp.float32),
            jax.ShapeDtypeStruct((N_PAD, 1), jnp.float32),
        ],
    )(h1, deg_parts)


def _tc2(s1, p1, dis, b1, W2):
    def body(s_ref, p_ref, dis_ref, bias_ref, w_ref, o_ref):
        dis = dis_ref[...]
        t = jnp.maximum(
            dis * (s_ref[0] + s_ref[1] + p_ref[...]) + bias_ref[...], 0.0)
        o_ref[...] = dis * jnp.dot(
            t, w_ref[...], preferred_element_type=jnp.float32)

    return pl.pallas_call(
        body,
        grid=(NB,),
        in_specs=[
            pl.BlockSpec((NC, RB, H1), lambda i: (0, i, 0)),
            pl.BlockSpec((RB, H1), lambda i: (i, 0)),
            pl.BlockSpec((RB, 1), lambda i: (i, 0)),
            pl.BlockSpec((1, H1), lambda i: (0, 0)),
            pl.BlockSpec((H1, H2), lambda i: (0, 0)),
        ],
        out_specs=pl.BlockSpec((RB, H2), lambda i: (i, 0)),
        out_shape=jax.ShapeDtypeStruct((N_PAD, H2), jnp.float32),
    )(s1, p1, dis, b1, W2)


def _tc3(s2, p2, dis, b2, batch_row, Wfc, bfc):
    def body(s_ref, p_ref, dis_ref, bias_ref, bt_ref, wfc_ref, bfc_ref,
             o_ref, sums, cnts):
        i = pl.program_id(0)

        @pl.when(i == 0)
        def _():
            sums[...] = jnp.zeros_like(sums)
            cnts[...] = jnp.zeros_like(cnts)

        dis = dis_ref[...]
        sv = s_ref[...]
        t = jnp.maximum(
            dis * (sv[0, :, :H2] + sv[1, :, :H2] + p_ref[...])
            + bias_ref[...], 0.0)
        b = bt_ref[0]
        gid = lax.broadcasted_iota(jnp.int32, (N_GRAPHS, RB), 0)
        col = lax.broadcasted_iota(jnp.int32, (N_GRAPHS, RB), 1) + i * RB
        oh = ((b == gid) & (col < N_NODES)).astype(jnp.float32)
        sums[...] += lax.dot_general(
            oh, t, (((1,), (0,)), ((), ())),
            preferred_element_type=jnp.float32,
            precision=lax.Precision.HIGHEST)
        cnts[...] += lax.dot_general(
            oh, jnp.ones((RB, 1), jnp.float32), (((1,), (0,)), ((), ())),
            preferred_element_type=jnp.float32,
            precision=lax.Precision.HIGHEST)

        @pl.when(i == NB - 1)
        def _():
            g = sums[...] / jnp.maximum(cnts[...], 1.0)
            logits = jnp.dot(
                g, wfc_ref[...], preferred_element_type=jnp.float32,
                precision=lax.Precision.HIGHEST) + bfc_ref[...]
            m = jnp.max(logits, axis=1, keepdims=True)
            e = jnp.exp(logits - m)
            o_ref[...] = e / jnp.sum(e, axis=1, keepdims=True)

    return pl.pallas_call(
        body,
        grid=(NB,),
        in_specs=[
            pl.BlockSpec((NC, RB, 128), lambda i: (0, i, 0)),
            pl.BlockSpec((RB, H2), lambda i: (i, 0)),
            pl.BlockSpec((RB, 1), lambda i: (i, 0)),
            pl.BlockSpec((1, H2), lambda i: (0, 0)),
            pl.BlockSpec((1, 1, RB), lambda i: (0, 0, i)),
            pl.BlockSpec((H2, D_OUT), lambda i: (0, 0)),
            pl.BlockSpec((1, D_OUT), lambda i: (0, 0)),
        ],
        out_specs=pl.BlockSpec((N_GRAPHS, D_OUT), lambda i: (0, 0)),
        out_shape=jax.ShapeDtypeStruct((N_GRAPHS, D_OUT), jnp.float32),
        scratch_shapes=[
            pltpu.VMEM((N_GRAPHS, H2), jnp.float32),
            pltpu.VMEM((N_GRAPHS, 1), jnp.float32),
        ],
    )(s2, p2, dis, b2, batch_row, Wfc, bfc)


def kernel(x, edge_index, batch, W1, b1, W2, b2, Wfc, bfc):
    src_r = edge_index[0].astype(jnp.int32).reshape(NCH_TOT, CHUNK)
    dst_r = edge_index[1].astype(jnp.int32).reshape(NCH_TOT, CHUNK)
    batch_row = batch.astype(jnp.int32).reshape(1, 1, N_NODES)

    deg_parts = _sc_degree(dst_r)
    h1 = _tc_h1(x, W1)
    p1, dis = _tc_scale(h1, deg_parts)
    s1 = _sc_spmm(p1, src_r, dst_r, H1, NBUF1, H1)
    p2 = _tc2(s1, p1, dis, b1.reshape(1, H1), W2)
    s2 = _sc_spmm(p2, src_r, dst_r, H2, NBUF2, 128)
    return _tc3(s2, p2, dis, b2.reshape(1, H2), batch_row,
                Wfc, bfc.reshape(1, D_OUT))


# speedup vs baseline: 43.9837x; 1.0594x over previous
import functools

import jax
import jax.numpy as jnp
from jax import lax
from jax.experimental import pallas as pl
from jax.experimental.pallas import tpu as pltpu
from jax.experimental.pallas import tpu_sc as plsc

N_NODES = 10000
N_GRAPHS = 64
D_IN = 128
H1 = 128
H2 = 64
D_OUT = 10
N_EDGES = 320000

NC = 2
NS = 16
NW = NC * NS
CHUNK = 128
NCH_TOT = N_EDGES // CHUNK
BASE_CH = NCH_TOT // NW
EXTRA = NCH_TOT % NW
MAX_CH = BASE_CH + 1
HALF = 40
N_PAD = 10240
ROWS_PT = N_PAD // NS
NBUF1 = 2
NBUF2 = 4
RB = 1280
NB = N_PAD // RB


def _worker_range(wid):
    start = wid * BASE_CH + jnp.minimum(wid, EXTRA)
    nch = BASE_CH + jnp.where(wid < EXTRA, 1, 0)
    return start, nch


def _load_chunks(ei_hbm, comp, start, wid, buf):
    pltpu.sync_copy(ei_hbm.at[pl.ds(start, BASE_CH), comp],
                    buf.at[pl.ds(0, BASE_CH)])

    @pl.when(wid < EXTRA)
    def _():
        pltpu.sync_copy(ei_hbm.at[pl.ds(start + BASE_CH, 1), comp],
                        buf.at[pl.ds(BASE_CH, 1)])


def _sc_degree(dst_r):
    mesh = plsc.VectorSubcoreMesh(core_axis_name="c", subcore_axis_name="s")

    @functools.partial(
        pl.kernel,
        out_type=jax.ShapeDtypeStruct((NC, N_PAD), jnp.float32),
        mesh=mesh,
        scratch_types=[
            pltpu.VMEM((MAX_CH, CHUNK), jnp.int32),
            pltpu.VMEM((CHUNK,), jnp.float32),
            pltpu.VMEM((ROWS_PT,), jnp.float32),
            pltpu.VMEM_SHARED((N_PAD,), jnp.float32),
        ],
        compiler_params=pltpu.CompilerParams(use_tc_tiling_on_sc=False),
    )
    def deg_kernel(ei_hbm, out_hbm, idx_v, ones_v, stage_v, acc_sh):
        c = lax.axis_index("c")
        s = lax.axis_index("s")
        wid = s * NC + c
        start, nch = _worker_range(wid)

        def zfill(i, _):
            stage_v[pl.ds(i * 16, 16)] = jnp.zeros((16,), jnp.float32)
            return 0

        lax.fori_loop(0, ROWS_PT // 16, zfill, 0)

        def ofill(i, _):
            ones_v[pl.ds(i * 16, 16)] = jnp.ones((16,), jnp.float32)
            return 0

        lax.fori_loop(0, CHUNK // 16, ofill, 0)

        pltpu.sync_copy(stage_v, acc_sh.at[pl.ds(s * ROWS_PT, ROWS_PT)])
        _load_chunks(ei_hbm, 1, start, wid, idx_v)
        plsc.subcore_barrier()

        def edge_body(j, _):
            pltpu.sync_copy(ones_v, acc_sh.at[idx_v.at[j]], add=True)
            return 0

        lax.fori_loop(0, nch, edge_body, 0)
        plsc.subcore_barrier()
        pltpu.sync_copy(acc_sh.at[pl.ds(s * ROWS_PT, ROWS_PT)], stage_v)
        pltpu.sync_copy(stage_v, out_hbm.at[c, pl.ds(s * ROWS_PT, ROWS_PT)])

    return deg_kernel(dst_r)


def _sc_spmm(p, src_r, d, nbuf, out_width):
    mesh = plsc.VectorSubcoreMesh(core_axis_name="c", subcore_axis_name="s")

    @functools.partial(
        pl.kernel,
        out_type=jax.ShapeDtypeStruct((NC, N_PAD, out_width), jnp.float32),
        mesh=mesh,
        scratch_types=[
            pltpu.VMEM((MAX_CH, CHUNK), jnp.int32),
            pltpu.VMEM((HALF, CHUNK), jnp.int32),
            pltpu.VMEM((nbuf, CHUNK, d), jnp.float32),
            pltpu.VMEM_SHARED((N_PAD, d), jnp.float32),
            pltpu.SemaphoreType.DMA((nbuf,)),
        ],
        compiler_params=pltpu.CompilerParams(use_tc_tiling_on_sc=False),
    )
    def spmm_kernel(p_hbm, ei_hbm, out_hbm, sidx, didx, rows,
                    acc_sh, gsem):
        c = lax.axis_index("c")
        s = lax.axis_index("s")
        wid = s * NC + c
        start, nch = _worker_range(wid)

        def zrow(i, _):
            for q in range(d // 16):
                rows[0, i, pl.ds(q * 16, 16)] = jnp.zeros((16,), jnp.float32)
            return 0

        lax.fori_loop(0, CHUNK, zrow, 0)
        for t in range(ROWS_PT // CHUNK):
            pltpu.sync_copy(
                rows.at[0], acc_sh.at[pl.ds(s * ROWS_PT + t * CHUNK, CHUNK)])
        _load_chunks(ei_hbm, 0, start, wid, sidx)
        for b in range(nbuf):
            pltpu.async_copy(p_hbm.at[sidx.at[b]], rows.at[b], gsem.at[b])
        plsc.subcore_barrier()

        def gather_wait(j, b):
            pltpu.make_async_copy(
                p_hbm.at[sidx.at[j]], rows.at[b], gsem.at[b]).wait()

        def issue_ahead(j, b):
            @pl.when(j + nbuf < nch)
            def _():
                pltpu.async_copy(
                    p_hbm.at[sidx.at[j + nbuf]], rows.at[b], gsem.at[b])

        pltpu.sync_copy(ei_hbm.at[pl.ds(start, HALF), 1], didx)

        def body0(jj, _):
            for b in range(nbuf):
                j = jj * nbuf + b
                gather_wait(j, b)
                pltpu.sync_copy(rows.at[b], acc_sh.at[didx.at[j]], add=True)
                issue_ahead(j, b)
            return 0

        lax.fori_loop(0, HALF // nbuf, body0, 0)

        pltpu.sync_copy(ei_hbm.at[pl.ds(start + nch - HALF, HALF), 1], didx)

        def body1(j, _):
            b = lax.rem(j, nbuf)
            gather_wait(j, b)
            pltpu.sync_copy(
                rows.at[b], acc_sh.at[didx.at[j - (nch - HALF)]], add=True)
            issue_ahead(j, b)
            return 0

        lax.fori_loop(HALF, nch, body1, 0)

        plsc.subcore_barrier()
        for t in range(ROWS_PT // CHUNK):
            r0 = s * ROWS_PT + t * CHUNK
            pltpu.sync_copy(acc_sh.at[pl.ds(r0, CHUNK)], rows.at[0])
            if out_width == d:
                pltpu.sync_copy(rows.at[0], out_hbm.at[c, pl.ds(r0, CHUNK)])
            else:
                pltpu.sync_copy(
                    rows.at[0], out_hbm.at[c, pl.ds(r0, CHUNK), pl.ds(0, d)])

    return spmm_kernel(p, src_r)


def _tc_h1(x, W1):
    def body(x_ref, w_ref, h_ref):
        i = pl.program_id(0)
        h = jnp.dot(x_ref[...], w_ref[...], preferred_element_type=jnp.float32)
        rows = lax.broadcasted_iota(jnp.int32, (RB, 1), 0) + i * RB
        h_ref[...] = jnp.where(rows < N_NODES, h, 0.0)

    return pl.pallas_call(
        body,
        grid=(NB,),
        in_specs=[
            pl.BlockSpec((RB, D_IN), lambda i: (i, 0)),
            pl.BlockSpec((D_IN, H1), lambda i: (0, 0)),
        ],
        out_specs=pl.BlockSpec((RB, H1), lambda i: (i, 0)),
        out_shape=jax.ShapeDtypeStruct((N_PAD, H1), jnp.float32),
    )(x, W1)


def _tc_scale(h1, deg_parts):
    def body(h_ref, dp_ref, p_ref, dis_ref):
        deg = dp_ref[0] + dp_ref[1] + 1.0
        dis = jnp.reshape(lax.rsqrt(deg), (RB, 1))
        p_ref[...] = h_ref[...] * dis
        dis_ref[...] = dis

    return pl.pallas_call(
        body,
        grid=(NB,),
        in_specs=[
            pl.BlockSpec((RB, H1), lambda i: (i, 0)),
            pl.BlockSpec((NC, RB), lambda i: (0, i)),
        ],
        out_specs=[
            pl.BlockSpec((RB, H1), lambda i: (i, 0)),
            pl.BlockSpec((RB, 1), lambda i: (i, 0)),
        ],
        out_shape=[
            jax.ShapeDtypeStruct((N_PAD, H1), jnp.float32),
            jax.ShapeDtypeStruct((N_PAD, 1), jnp.float32),
        ],
    )(h1, deg_parts)


def _tc2(s1, p1, dis, b1, W2):
    def body(s_ref, p_ref, dis_ref, bias_ref, w_ref, o_ref):
        dis = dis_ref[...]
        t = jnp.maximum(
            dis * (s_ref[0] + s_ref[1] + p_ref[...]) + bias_ref[...], 0.0)
        o_ref[...] = dis * jnp.dot(
            t, w_ref[...], preferred_element_type=jnp.float32)

    return pl.pallas_call(
        body,
        grid=(NB,),
        in_specs=[
            pl.BlockSpec((NC, RB, H1), lambda i: (0, i, 0)),
            pl.BlockSpec((RB, H1), lambda i: (i, 0)),
            pl.BlockSpec((RB, 1), lambda i: (i, 0)),
            pl.BlockSpec((1, H1), lambda i: (0, 0)),
            pl.BlockSpec((H1, H2), lambda i: (0, 0)),
        ],
        out_specs=pl.BlockSpec((RB, H2), lambda i: (i, 0)),
        out_shape=jax.ShapeDtypeStruct((N_PAD, H2), jnp.float32),
    )(s1, p1, dis, b1, W2)


def _tc3(s2, p2, dis, b2, batch_row, Wfc, bfc):
    def body(s_ref, p_ref, dis_ref, bias_ref, bt_ref, wfc_ref, bfc_ref,
             o_ref, sums, cnts):
        i = pl.program_id(0)

        @pl.when(i == 0)
        def _():
            sums[...] = jnp.zeros_like(sums)
            cnts[...] = jnp.zeros_like(cnts)

        dis = dis_ref[...]
        sv = s_ref[...]
        t = jnp.maximum(
            dis * (sv[0, :, :H2] + sv[1, :, :H2] + p_ref[...])
            + bias_ref[...], 0.0)
        b = bt_ref[0]
        gid = lax.broadcasted_iota(jnp.int32, (N_GRAPHS, RB), 0)
        col = lax.broadcasted_iota(jnp.int32, (N_GRAPHS, RB), 1) + i * RB
        oh = ((b == gid) & (col < N_NODES)).astype(jnp.float32)
        sums[...] += lax.dot_general(
            oh, t, (((1,), (0,)), ((), ())),
            preferred_element_type=jnp.float32,
            precision=lax.Precision.HIGHEST)
        cnts[...] += lax.dot_general(
            oh, jnp.ones((RB, 1), jnp.float32), (((1,), (0,)), ((), ())),
            preferred_element_type=jnp.float32,
            precision=lax.Precision.HIGHEST)

        @pl.when(i == NB - 1)
        def _():
            g = sums[...] / jnp.maximum(cnts[...], 1.0)
            logits = jnp.dot(
                g, wfc_ref[...], preferred_element_type=jnp.float32,
                precision=lax.Precision.HIGHEST) + bfc_ref[...]
            m = jnp.max(logits, axis=1, keepdims=True)
            e = jnp.exp(logits - m)
            o_ref[...] = e / jnp.sum(e, axis=1, keepdims=True)

    return pl.pallas_call(
        body,
        grid=(NB,),
        in_specs=[
            pl.BlockSpec((NC, RB, 128), lambda i: (0, i, 0)),
            pl.BlockSpec((RB, H2), lambda i: (i, 0)),
            pl.BlockSpec((RB, 1), lambda i: (i, 0)),
            pl.BlockSpec((1, H2), lambda i: (0, 0)),
            pl.BlockSpec((1, 1, RB), lambda i: (0, 0, i)),
            pl.BlockSpec((H2, D_OUT), lambda i: (0, 0)),
            pl.BlockSpec((1, D_OUT), lambda i: (0, 0)),
        ],
        out_specs=pl.BlockSpec((N_GRAPHS, D_OUT), lambda i: (0, 0)),
        out_shape=jax.ShapeDtypeStruct((N_GRAPHS, D_OUT), jnp.float32),
        scratch_shapes=[
            pltpu.VMEM((N_GRAPHS, H2), jnp.float32),
            pltpu.VMEM((N_GRAPHS, 1), jnp.float32),
        ],
    )(s2, p2, dis, b2, batch_row, Wfc, bfc)


def kernel(x, edge_index, batch, W1, b1, W2, b2, Wfc, bfc):
    ei = jnp.transpose(
        edge_index.astype(jnp.int32).reshape(2, NCH_TOT, CHUNK), (1, 0, 2))
    batch_row = batch.astype(jnp.int32).reshape(1, 1, N_NODES)

    deg_parts = _sc_degree(ei)
    h1 = _tc_h1(x, W1)
    p1, dis = _tc_scale(h1, deg_parts)
    s1 = _sc_spmm(p1, ei, H1, NBUF1, H1)
    p2 = _tc2(s1, p1, dis, b1.reshape(1, H1), W2)
    s2 = _sc_spmm(p2, ei, H2, NBUF2, 128)
    return _tc3(s2, p2, dis, b2.reshape(1, H2), batch_row,
                Wfc, bfc.reshape(1, D_OUT))


# speedup vs baseline: 44.7592x; 1.0176x over previous
import functools

import jax
import jax.numpy as jnp
from jax import lax
from jax.experimental import pallas as pl
from jax.experimental.pallas import tpu as pltpu
from jax.experimental.pallas import tpu_sc as plsc

N_NODES = 10000
N_GRAPHS = 64
D_IN = 128
H1 = 128
H2 = 64
D_OUT = 10
N_EDGES = 320000

NC = 2
NS = 16
NW = NC * NS
CHUNK = 128
NCH_TOT = N_EDGES // CHUNK
BASE_CH = NCH_TOT // NW
EXTRA = NCH_TOT % NW
MAX_CH = BASE_CH + 1
HALF = 40
N_PAD = 10240
ROWS_PT = N_PAD // NS
NBUF1 = 2
NBUF2 = 4
RB = 2560
NB = N_PAD // RB


def _worker_range(wid):
    start = wid * BASE_CH + jnp.minimum(wid, EXTRA)
    nch = BASE_CH + jnp.where(wid < EXTRA, 1, 0)
    return start, nch


def _load_chunks(ei_hbm, comp, start, wid, buf):
    pltpu.sync_copy(ei_hbm.at[pl.ds(start, BASE_CH), comp],
                    buf.at[pl.ds(0, BASE_CH)])

    @pl.when(wid < EXTRA)
    def _():
        pltpu.sync_copy(ei_hbm.at[pl.ds(start + BASE_CH, 1), comp],
                        buf.at[pl.ds(BASE_CH, 1)])


def _sc_degree(dst_r):
    mesh = plsc.VectorSubcoreMesh(core_axis_name="c", subcore_axis_name="s")

    @functools.partial(
        pl.kernel,
        out_type=jax.ShapeDtypeStruct((NC, N_PAD), jnp.float32),
        mesh=mesh,
        scratch_types=[
            pltpu.VMEM((MAX_CH, CHUNK), jnp.int32),
            pltpu.VMEM((CHUNK,), jnp.float32),
            pltpu.VMEM((ROWS_PT,), jnp.float32),
            pltpu.VMEM_SHARED((N_PAD,), jnp.float32),
        ],
        compiler_params=pltpu.CompilerParams(use_tc_tiling_on_sc=False),
    )
    def deg_kernel(ei_hbm, out_hbm, idx_v, ones_v, stage_v, acc_sh):
        c = lax.axis_index("c")
        s = lax.axis_index("s")
        wid = s * NC + c
        start, nch = _worker_range(wid)

        def zfill(i, _):
            stage_v[pl.ds(i * 16, 16)] = jnp.zeros((16,), jnp.float32)
            return 0

        lax.fori_loop(0, ROWS_PT // 16, zfill, 0)

        def ofill(i, _):
            ones_v[pl.ds(i * 16, 16)] = jnp.ones((16,), jnp.float32)
            return 0

        lax.fori_loop(0, CHUNK // 16, ofill, 0)

        pltpu.sync_copy(stage_v, acc_sh.at[pl.ds(s * ROWS_PT, ROWS_PT)])
        _load_chunks(ei_hbm, 1, start, wid, idx_v)
        plsc.subcore_barrier()

        def edge_body(j, _):
            pltpu.sync_copy(ones_v, acc_sh.at[idx_v.at[j]], add=True)
            return 0

        lax.fori_loop(0, nch, edge_body, 0)
        plsc.subcore_barrier()
        pltpu.sync_copy(acc_sh.at[pl.ds(s * ROWS_PT, ROWS_PT)], stage_v)
        pltpu.sync_copy(stage_v, out_hbm.at[c, pl.ds(s * ROWS_PT, ROWS_PT)])

    return deg_kernel(dst_r)


def _sc_spmm(p, src_r, d, nbuf, out_width):
    mesh = plsc.VectorSubcoreMesh(core_axis_name="c", subcore_axis_name="s")

    @functools.partial(
        pl.kernel,
        out_type=jax.ShapeDtypeStruct((NC, N_PAD, out_width), jnp.float32),
        mesh=mesh,
        scratch_types=[
            pltpu.VMEM((MAX_CH, CHUNK), jnp.int32),
            pltpu.VMEM((HALF, CHUNK), jnp.int32),
            pltpu.VMEM((nbuf, CHUNK, d), jnp.float32),
            pltpu.VMEM_SHARED((N_PAD, d), jnp.float32),
            pltpu.SemaphoreType.DMA((nbuf,)),
        ],
        compiler_params=pltpu.CompilerParams(use_tc_tiling_on_sc=False),
    )
    def spmm_kernel(p_hbm, ei_hbm, out_hbm, sidx, didx, rows,
                    acc_sh, gsem):
        c = lax.axis_index("c")
        s = lax.axis_index("s")
        wid = s * NC + c
        start, nch = _worker_range(wid)

        def zrow(i, _):
            for q in range(d // 16):
                rows[0, i, pl.ds(q * 16, 16)] = jnp.zeros((16,), jnp.float32)
            return 0

        lax.fori_loop(0, CHUNK, zrow, 0)
        for t in range(ROWS_PT // CHUNK):
            pltpu.sync_copy(
                rows.at[0], acc_sh.at[pl.ds(s * ROWS_PT + t * CHUNK, CHUNK)])
        _load_chunks(ei_hbm, 0, start, wid, sidx)
        for b in range(nbuf):
            pltpu.async_copy(p_hbm.at[sidx.at[b]], rows.at[b], gsem.at[b])
        plsc.subcore_barrier()

        def gather_wait(j, b):
            pltpu.make_async_copy(
                p_hbm.at[sidx.at[j]], rows.at[b], gsem.at[b]).wait()

        def issue_ahead(j, b):
            @pl.when(j + nbuf < nch)
            def _():
                pltpu.async_copy(
                    p_hbm.at[sidx.at[j + nbuf]], rows.at[b], gsem.at[b])

        pltpu.sync_copy(ei_hbm.at[pl.ds(start, HALF), 1], didx)

        def body0(jj, _):
            for b in range(nbuf):
                j = jj * nbuf + b
                gather_wait(j, b)
                pltpu.sync_copy(rows.at[b], acc_sh.at[didx.at[j]], add=True)
                issue_ahead(j, b)
            return 0

        lax.fori_loop(0, HALF // nbuf, body0, 0)

        pltpu.sync_copy(ei_hbm.at[pl.ds(start + nch - HALF, HALF), 1], didx)

        def body1(j, _):
            b = lax.rem(j, nbuf)
            gather_wait(j, b)
            pltpu.sync_copy(
                rows.at[b], acc_sh.at[didx.at[j - (nch - HALF)]], add=True)
            issue_ahead(j, b)
            return 0

        lax.fori_loop(HALF, nch, body1, 0)

        plsc.subcore_barrier()
        for t in range(ROWS_PT // CHUNK):
            r0 = s * ROWS_PT + t * CHUNK
            pltpu.sync_copy(acc_sh.at[pl.ds(r0, CHUNK)], rows.at[0])
            if out_width == d:
                pltpu.sync_copy(rows.at[0], out_hbm.at[c, pl.ds(r0, CHUNK)])
            else:
                pltpu.sync_copy(
                    rows.at[0], out_hbm.at[c, pl.ds(r0, CHUNK), pl.ds(0, d)])

    return spmm_kernel(p, src_r)


def _tc_h1(x, W1):
    def body(x_ref, w_ref, h_ref):
        i = pl.program_id(0)
        h = jnp.dot(x_ref[...], w_ref[...], preferred_element_type=jnp.float32)
        rows = lax.broadcasted_iota(jnp.int32, (RB, 1), 0) + i * RB
        h_ref[...] = jnp.where(rows < N_NODES, h, 0.0)

    return pl.pallas_call(
        body,
        grid=(NB,),
        in_specs=[
            pl.BlockSpec((RB, D_IN), lambda i: (i, 0)),
            pl.BlockSpec((D_IN, H1), lambda i: (0, 0)),
        ],
        out_specs=pl.BlockSpec((RB, H1), lambda i: (i, 0)),
        out_shape=jax.ShapeDtypeStruct((N_PAD, H1), jnp.float32),
    )(x, W1)


def _tc_scale(h1, deg_parts):
    def body(h_ref, dp_ref, p_ref, dis_ref):
        deg = dp_ref[0] + dp_ref[1] + 1.0
        dis = jnp.reshape(lax.rsqrt(deg), (RB, 1))
        p_ref[...] = h_ref[...] * dis
        dis_ref[...] = dis

    return pl.pallas_call(
        body,
        grid=(NB,),
        in_specs=[
            pl.BlockSpec((RB, H1), lambda i: (i, 0)),
            pl.BlockSpec((NC, RB), lambda i: (0, i)),
        ],
        out_specs=[
            pl.BlockSpec((RB, H1), lambda i: (i, 0)),
            pl.BlockSpec((RB, 1), lambda i: (i, 0)),
        ],
        out_shape=[
            jax.ShapeDtypeStruct((N_PAD, H1), jnp.float32),
            jax.ShapeDtypeStruct((N_PAD, 1), jnp.float32),
        ],
    )(h1, deg_parts)


def _tc2(s1, p1, dis, b1, W2):
    def body(s_ref, p_ref, dis_ref, bias_ref, w_ref, o_ref):
        dis = dis_ref[...]
        t = jnp.maximum(
            dis * (s_ref[0] + s_ref[1] + p_ref[...]) + bias_ref[...], 0.0)
        o_ref[...] = dis * jnp.dot(
            t, w_ref[...], preferred_element_type=jnp.float32)

    return pl.pallas_call(
        body,
        grid=(NB,),
        in_specs=[
            pl.BlockSpec((NC, RB, H1), lambda i: (0, i, 0)),
            pl.BlockSpec((RB, H1), lambda i: (i, 0)),
            pl.BlockSpec((RB, 1), lambda i: (i, 0)),
            pl.BlockSpec((1, H1), lambda i: (0, 0)),
            pl.BlockSpec((H1, H2), lambda i: (0, 0)),
        ],
        out_specs=pl.BlockSpec((RB, H2), lambda i: (i, 0)),
        out_shape=jax.ShapeDtypeStruct((N_PAD, H2), jnp.float32),
    )(s1, p1, dis, b1, W2)


def _tc3(s2, p2, dis, b2, batch_row, Wfc, bfc):
    def body(s_ref, p_ref, dis_ref, bias_ref, bt_ref, wfc_ref, bfc_ref,
             o_ref, sums, cnts):
        i = pl.program_id(0)

        @pl.when(i == 0)
        def _():
            sums[...] = jnp.zeros_like(sums)
            cnts[...] = jnp.zeros_like(cnts)

        dis = dis_ref[...]
        sv = s_ref[...]
        t = jnp.maximum(
            dis * (sv[0, :, :H2] + sv[1, :, :H2] + p_ref[...])
            + bias_ref[...], 0.0)
        b = bt_ref[0]
        gid = lax.broadcasted_iota(jnp.int32, (N_GRAPHS, RB), 0)
        col = lax.broadcasted_iota(jnp.int32, (N_GRAPHS, RB), 1) + i * RB
        oh = ((b == gid) & (col < N_NODES)).astype(jnp.float32)
        sums[...] += lax.dot_general(
            oh, t, (((1,), (0,)), ((), ())),
            preferred_element_type=jnp.float32,
            precision=lax.Precision.HIGHEST)
        cnts[...] += lax.dot_general(
            oh, jnp.ones((RB, 1), jnp.float32), (((1,), (0,)), ((), ())),
            preferred_element_type=jnp.float32,
            precision=lax.Precision.HIGHEST)

        @pl.when(i == NB - 1)
        def _():
            g = sums[...] / jnp.maximum(cnts[...], 1.0)
            logits = jnp.dot(
                g, wfc_ref[...], preferred_element_type=jnp.float32,
                precision=lax.Precision.HIGHEST) + bfc_ref[...]
            m = jnp.max(logits, axis=1, keepdims=True)
            e = jnp.exp(logits - m)
            o_ref[...] = e / jnp.sum(e, axis=1, keepdims=True)

    return pl.pallas_call(
        body,
        grid=(NB,),
        in_specs=[
            pl.BlockSpec((NC, RB, 128), lambda i: (0, i, 0)),
            pl.BlockSpec((RB, H2), lambda i: (i, 0)),
            pl.BlockSpec((RB, 1), lambda i: (i, 0)),
            pl.BlockSpec((1, H2), lambda i: (0, 0)),
            pl.BlockSpec((1, 1, RB), lambda i: (0, 0, i)),
            pl.BlockSpec((H2, D_OUT), lambda i: (0, 0)),
            pl.BlockSpec((1, D_OUT), lambda i: (0, 0)),
        ],
        out_specs=pl.BlockSpec((N_GRAPHS, D_OUT), lambda i: (0, 0)),
        out_shape=jax.ShapeDtypeStruct((N_GRAPHS, D_OUT), jnp.float32),
        scratch_shapes=[
            pltpu.VMEM((N_GRAPHS, H2), jnp.float32),
            pltpu.VMEM((N_GRAPHS, 1), jnp.float32),
        ],
    )(s2, p2, dis, b2, batch_row, Wfc, bfc)


def kernel(x, edge_index, batch, W1, b1, W2, b2, Wfc, bfc):
    ei = jnp.transpose(
        edge_index.astype(jnp.int32).reshape(2, NCH_TOT, CHUNK), (1, 0, 2))
    batch_row = batch.astype(jnp.int32).reshape(1, 1, N_NODES)

    deg_parts = _sc_degree(ei)
    h1 = _tc_h1(x, W1)
    p1, dis = _tc_scale(h1, deg_parts)
    s1 = _sc_spmm(p1, ei, H1, NBUF1, H1)
    p2 = _tc2(s1, p1, dis, b1.reshape(1, H1), W2)
    s2 = _sc_spmm(p2, ei, H2, NBUF2, 128)
    return _tc3(s2, p2, dis, b2.reshape(1, H2), batch_row,
                Wfc, bfc.reshape(1, D_OUT))
